# baseline (speedup 1.0000x reference)
import functools

import numpy as np
import jax
import jax.numpy as jnp
from jax import lax
from jax.experimental import pallas as pl
from jax.experimental.pallas import tpu as pltpu
from jax.experimental.pallas import tpu_sc as plsc

_NUM_CLASSES = 80
_K = 10
_EPSILON = 1e-09


def _atan_pos(x):
    t38 = 2.414213562373095
    t8 = 0.4142135623730950
    use_big = x > t38
    use_mid = (x > t8) & (~use_big)
    arg = jnp.where(use_big, -1.0 / x, jnp.where(use_mid, (x - 1.0) / (x + 1.0), x))
    base = jnp.where(use_big, jnp.float32(np.pi / 2),
                     jnp.where(use_mid, jnp.float32(np.pi / 4), jnp.float32(0.0)))
    z = arg * arg
    poly = (((8.05374449538e-2 * z - 1.38776856032e-1) * z
             + 1.99777106478e-1) * z - 3.33329491539e-1) * z * arg + arg
    return base + poly


def _dense_kernel(scores_t_ref, decode_t_ref, anchors_t_ref, gt_labels_ref,
                  gt_bboxes_ref, gt_mask_ref, align_ref, ciou_ref,
                  gsum_ref):
    f32 = jnp.float32
    C, A = scores_t_ref.shape[1], scores_t_ref.shape[2]
    G = gt_bboxes_ref.shape[1]

    scores_t = scores_t_ref[0]
    dec = decode_t_ref[0]
    anc = anchors_t_ref[...]
    gtb = gt_bboxes_ref[0]
    labels = gt_labels_ref[0]
    gmask = gt_mask_ref[0]

    d_x1 = dec[0:1, :]
    d_y1 = dec[1:2, :]
    d_x2 = dec[2:3, :]
    d_y2 = dec[3:4, :]
    a_x = anc[0:1, :]
    a_y = anc[1:2, :]
    g_x1 = gtb[:, 0:1]
    g_y1 = gtb[:, 1:2]
    g_x2 = gtb[:, 2:3]
    g_y2 = gtb[:, 3:4]

    cls = jnp.maximum(labels, 0)
    onehot_cls = jnp.where(
        gmask > 0.0,
        (lax.broadcasted_iota(jnp.int32, (G, C), 1) == cls).astype(f32),
        0.0)
    bbox_scores = lax.dot_general(
        onehot_cls, scores_t, (((1,), (0,)), ((), ())),
        preferred_element_type=f32, precision=lax.Precision.HIGHEST)

    eps = 1e-9
    x1 = jnp.maximum(g_x1, d_x1)
    y1 = jnp.maximum(g_y1, d_y1)
    x2 = jnp.minimum(g_x2, d_x2)
    y2 = jnp.minimum(g_y2, d_y2)
    inter = jnp.maximum(x2 - x1, 0.0) * jnp.maximum(y2 - y1, 0.0)
    w1 = g_x2 - g_x1
    h1 = g_y2 - g_y1
    w2 = d_x2 - d_x1
    h2 = d_y2 - d_y1
    union = w1 * h1 + w2 * h2 - inter
    iou = inter / (union + eps)
    cw = jnp.maximum(g_x2, d_x2) - jnp.minimum(g_x1, d_x1)
    ch = jnp.maximum(g_y2, d_y2) - jnp.minimum(g_y1, d_y1)
    c2 = cw * cw + ch * ch + eps
    dx = d_x1 + d_x2 - g_x1 - g_x2
    dy = d_y1 + d_y2 - g_y1 - g_y2
    rho2 = (dx * dx + dy * dy) * 0.25
    atan_g = _atan_pos(w1 / (h1 + eps))
    atan_d = _atan_pos(w2 / (h2 + eps))
    dv = atan_d - atan_g
    v = (4.0 / (np.pi ** 2)) * (dv * dv)
    alpha_t = v / (v - iou + 1.0 + eps)
    ciou = iou - (rho2 / c2 + v * alpha_t)

    o2 = ciou * ciou
    o6 = o2 * o2 * o2
    inbox = (g_x1 < a_x) & (g_y1 < a_y) & (g_x2 > a_x) & (g_y2 > a_y)
    align = jnp.where(inbox, jnp.sqrt(bbox_scores) * o6, 0.0)

    align_ref[0] = align
    ciou_ref[0] = ciou

    ngroups = A // _SC_GROUP_W
    ibf = jnp.where(inbox & (gmask > 0.0), 1.0, 0.0)
    seg = (lax.broadcasted_iota(jnp.int32, (A, ngroups), 0) // _SC_GROUP_W ==
           lax.broadcasted_iota(jnp.int32, (A, ngroups), 1)).astype(f32)
    gsum_ref[0] = lax.dot_general(
        ibf, seg, (((1,), (0,)), ((), ())),
        preferred_element_type=f32, precision=lax.Precision.HIGHEST)


_SC_ROWS_PER_WORKER = 32
_SC_GROUP_W = 80


def _sc_cand_body(align_hbm, gsum_hbm, out_hbm, row_v, gs_v, out_v):
    f32 = jnp.float32
    nchunks_per_group = _SC_GROUP_W // 16
    ngroups = align_hbm.shape[1] // _SC_GROUP_W
    wid = lax.axis_index("c") * 16 + lax.axis_index("s")
    base_row = wid * _SC_ROWS_PER_WORKER
    nrg = _SC_ROWS_PER_WORKER * ngroups

    pltpu.sync_copy(gsum_hbm.at[pl.ds(base_row * ngroups, nrg)],
                    gs_v.at[pl.ds(0, nrg)])

    neg1 = jnp.full((16,), -1.0, f32)

    def row_body(j, carry):
        rbase = j * (16 * _K)
        for i in range(_K):
            out_v[pl.ds(rbase + i * 16, 16)] = neg1
        pltpu.sync_copy(align_hbm.at[base_row + j], row_v)

        def group_body(g, carry2):
            gm = gs_v[pl.ds(j * ngroups + g, 16)][0]

            @pl.when(gm > 0.0)
            def _():
                tt = [out_v[pl.ds(rbase + i * 16, 16)] for i in range(_K)]
                for u in range(nchunks_per_group):
                    cur = row_v[pl.ds((g * nchunks_per_group + u) * 16, 16)]
                    for i in range(_K):
                        hi = jnp.maximum(tt[i], cur)
                        cur = jnp.minimum(tt[i], cur)
                        tt[i] = hi
                for i in range(_K):
                    out_v[pl.ds(rbase + i * 16, 16)] = tt[i]

            return carry2

        lax.fori_loop(0, ngroups, group_body, jnp.int32(0))
        return carry

    lax.fori_loop(0, _SC_ROWS_PER_WORKER, row_body, jnp.int32(0))
    pltpu.sync_copy(out_v, out_hbm.at[wid])


def _finish_kernel(align_ref, ciou_ref, cand_ref, gt_labels_ref,
                   gt_bboxes_ref, gt_dist_ref,
                   bbox_out_ref, cls_out_ref, dist_out_ref, fg_out_ref):
    f32 = jnp.float32
    G, A = align_ref.shape[1], align_ref.shape[2]
    C = cls_out_ref.shape[1]

    align = align_ref[0]
    ciou = ciou_ref[0]
    cand = cand_ref[0]
    gtb = gt_bboxes_ref[0]
    labels = gt_labels_ref[0]
    gdist = gt_dist_ref[0]

    work = cand
    thresh = jnp.full((G, 1), -1.0, f32)
    for _ in range(_K):
        thresh = jnp.max(work, axis=1, keepdims=True)
        work = jnp.where(work == thresh, -1.0, work)

    keep10 = (align >= thresh) & (align > 0.0)
    ov_m = jnp.where(keep10, ciou, 0.0)
    al_m = jnp.where(keep10, align, 0.0)

    best_ov = jnp.max(ov_m, axis=0, keepdims=True)
    iota_g = lax.broadcasted_iota(jnp.int32, (G, A), 0)
    best_g = jnp.min(jnp.where(ov_m == best_ov, iota_g, G), axis=0, keepdims=True)
    matched = best_ov > 0.0
    onehot_a = (iota_g == best_g).astype(f32)

    gvals = jnp.concatenate(
        [gtb, gdist, jnp.zeros((G, 3), f32)], axis=1)
    gathered = lax.dot_general(
        gvals, onehot_a, (((0,), (0,)), ((), ())),
        preferred_element_type=f32, precision=lax.Precision.HIGHEST)

    max_al = jnp.max(al_m, axis=1, keepdims=True)
    max_ov = jnp.max(ov_m, axis=1, keepdims=True)
    ratio = max_ov / (max_al + _EPSILON)
    norm = jnp.max(al_m * ratio, axis=0, keepdims=True)

    neg1 = jnp.float32(-1.0)
    bbox_out_ref[0] = jnp.where(matched, gathered[0:4, :], neg1)

    cls = jnp.maximum(labels, 0)
    onehot_cls = (lax.broadcasted_iota(jnp.int32, (G, C), 1) == cls).astype(f32)
    sel_scaled = onehot_a * jnp.where(matched, norm, 0.0)
    cls_out_ref[0] = lax.dot_general(
        onehot_cls, sel_scaled, (((0,), (0,)), ((), ())),
        preferred_element_type=f32, precision=lax.Precision.HIGHEST)

    dist_out_ref[0] = jnp.where(matched, gathered[4:5, :], neg1) * norm
    fg_out_ref[0] = jnp.ones((1, A), f32)


def kernel(scores, decode_bboxes, distances, anchors, gt_labels, gt_bboxes,
           gt_distances, gt_mask):
    del distances
    B, A, C = scores.shape
    G = gt_labels.shape[1]

    scores_t = jnp.transpose(scores, (0, 2, 1))
    decode_t = jnp.transpose(decode_bboxes, (0, 2, 1))
    anchors_t = jnp.transpose(anchors, (1, 0))
    labels3 = gt_labels.reshape(B, G, 1)
    gdist3 = gt_distances.reshape(B, G, 1)
    gmask3 = gt_mask.astype(jnp.float32)

    def row_spec(shape):
        return pl.BlockSpec((1,) + shape, lambda b: (b, 0, 0))

    ngroups = A // _SC_GROUP_W

    align, ciou, gsum = pl.pallas_call(
        _dense_kernel,
        grid=(B,),
        in_specs=[
            row_spec((C, A)),
            row_spec((4, A)),
            pl.BlockSpec((2, A), lambda b: (0, 0)),
            row_spec((G, 1)),
            row_spec((G, 4)),
            row_spec((G, 1)),
        ],
        out_specs=(row_spec((G, A)), row_spec((G, A)),
                   row_spec((G, ngroups))),
        out_shape=(
            jax.ShapeDtypeStruct((B, G, A), jnp.float32),
            jax.ShapeDtypeStruct((B, G, A), jnp.float32),
            jax.ShapeDtypeStruct((B, G, ngroups), jnp.float32),
        ),
    )(scores_t, decode_t, anchors_t, labels3, gt_bboxes, gmask3)

    mesh = plsc.VectorSubcoreMesh(core_axis_name="c", subcore_axis_name="s")
    sc_call = functools.partial(
        pl.kernel, mesh=mesh,
        out_type=jax.ShapeDtypeStruct(
            (32, _SC_ROWS_PER_WORKER * 16 * _K), jnp.float32),
        scratch_types=[
            pltpu.VMEM((A,), jnp.float32),
            pltpu.VMEM((_SC_ROWS_PER_WORKER * ngroups + 16,), jnp.float32),
            pltpu.VMEM((_SC_ROWS_PER_WORKER * 16 * _K,), jnp.float32),
        ],
    )(_sc_cand_body)
    cand = sc_call(align.reshape(B * G, A),
                   gsum.reshape(B * G * ngroups))
    cand3 = cand.reshape(B, G, 16 * _K)

    out_shapes = (
        jax.ShapeDtypeStruct((B, 4, A), jnp.float32),
        jax.ShapeDtypeStruct((B, C, A), jnp.float32),
        jax.ShapeDtypeStruct((B, 1, A), jnp.float32),
        jax.ShapeDtypeStruct((B, 1, A), jnp.float32),
    )
    bbox_l, cls_oh, dist_l, fg = pl.pallas_call(
        _finish_kernel,
        grid=(B,),
        in_specs=[
            row_spec((G, A)),
            row_spec((G, A)),
            row_spec((G, 16 * _K)),
            row_spec((G, 1)),
            row_spec((G, 4)),
            row_spec((G, 1)),
        ],
        out_specs=(row_spec((4, A)), row_spec((C, A)),
                   row_spec((1, A)), row_spec((1, A))),
        out_shape=out_shapes,
    )(align, ciou, cand3, labels3, gt_bboxes, gdist3)

    bbox_labels = jnp.transpose(bbox_l, (0, 2, 1))
    class_labels_oh = jnp.transpose(cls_oh, (0, 2, 1))
    dist_labels = dist_l.reshape(B, A)
    fg_mask = fg.reshape(B, A)
    return bbox_labels, class_labels_oh, dist_labels, fg_mask


# speedup vs baseline: 1.3217x; 1.3217x over previous
import functools

import numpy as np
import jax
import jax.numpy as jnp
from jax import lax
from jax.experimental import pallas as pl
from jax.experimental.pallas import tpu as pltpu
from jax.experimental.pallas import tpu_sc as plsc

_NUM_CLASSES = 80
_K = 10
_EPSILON = 1e-09


def _atan_pos(x):
    t38 = 2.414213562373095
    t8 = 0.4142135623730950
    use_big = x > t38
    use_mid = (x > t8) & (~use_big)
    arg = jnp.where(use_big, -1.0 / x, jnp.where(use_mid, (x - 1.0) / (x + 1.0), x))
    base = jnp.where(use_big, jnp.float32(np.pi / 2),
                     jnp.where(use_mid, jnp.float32(np.pi / 4), jnp.float32(0.0)))
    z = arg * arg
    poly = (((8.05374449538e-2 * z - 1.38776856032e-1) * z
             + 1.99777106478e-1) * z - 3.33329491539e-1) * z * arg + arg
    return base + poly


def _dense_kernel(scores_t_ref, decode_t_ref, anchors_t_ref, gt_labels_ref,
                  gt_bboxes_ref, gt_mask_ref, align_ref, ciou_ref):
    f32 = jnp.float32
    C, A = scores_t_ref.shape[1], scores_t_ref.shape[2]
    G = gt_bboxes_ref.shape[1]

    scores_t = scores_t_ref[0]
    dec = decode_t_ref[0]
    anc = anchors_t_ref[...]
    gtb = gt_bboxes_ref[0]
    labels = gt_labels_ref[0]
    gmask = gt_mask_ref[0]

    d_x1 = dec[0:1, :]
    d_y1 = dec[1:2, :]
    d_x2 = dec[2:3, :]
    d_y2 = dec[3:4, :]
    a_x = anc[0:1, :]
    a_y = anc[1:2, :]
    g_x1 = gtb[:, 0:1]
    g_y1 = gtb[:, 1:2]
    g_x2 = gtb[:, 2:3]
    g_y2 = gtb[:, 3:4]

    cls = jnp.maximum(labels, 0)
    onehot_cls = jnp.where(
        gmask > 0.0,
        (lax.broadcasted_iota(jnp.int32, (G, C), 1) == cls).astype(f32),
        0.0)
    bbox_scores = lax.dot_general(
        onehot_cls, scores_t, (((1,), (0,)), ((), ())),
        preferred_element_type=f32, precision=lax.Precision.HIGHEST)

    eps = 1e-9
    x1 = jnp.maximum(g_x1, d_x1)
    y1 = jnp.maximum(g_y1, d_y1)
    x2 = jnp.minimum(g_x2, d_x2)
    y2 = jnp.minimum(g_y2, d_y2)
    inter = jnp.maximum(x2 - x1, 0.0) * jnp.maximum(y2 - y1, 0.0)
    w1 = g_x2 - g_x1
    h1 = g_y2 - g_y1
    w2 = d_x2 - d_x1
    h2 = d_y2 - d_y1
    union = w1 * h1 + w2 * h2 - inter
    iou = inter / (union + eps)
    cw = jnp.maximum(g_x2, d_x2) - jnp.minimum(g_x1, d_x1)
    ch = jnp.maximum(g_y2, d_y2) - jnp.minimum(g_y1, d_y1)
    c2 = cw * cw + ch * ch + eps
    dx = d_x1 + d_x2 - g_x1 - g_x2
    dy = d_y1 + d_y2 - g_y1 - g_y2
    rho2 = (dx * dx + dy * dy) * 0.25
    atan_g = _atan_pos(w1 / (h1 + eps))
    atan_d = _atan_pos(w2 / (h2 + eps))
    dv = atan_d - atan_g
    v = (4.0 / (np.pi ** 2)) * (dv * dv)
    alpha_t = v / (v - iou + 1.0 + eps)
    ciou = iou - (rho2 / c2 + v * alpha_t)

    o2 = ciou * ciou
    o6 = o2 * o2 * o2
    inbox = (g_x1 < a_x) & (g_y1 < a_y) & (g_x2 > a_x) & (g_y2 > a_y)
    align = jnp.where(inbox, jnp.sqrt(bbox_scores) * o6, 0.0)

    align_ref[0] = align
    ciou_ref[0] = ciou


_SC_ROWS_PER_WORKER = 32
_SC_GROUP_W = 80


def _sc_cand_body(align_hbm, out_hbm, row_v, out_v):
    f32 = jnp.float32
    nchunks = align_hbm.shape[1] // 16
    rows_per_worker = align_hbm.shape[0] // 32
    wid = lax.axis_index("c") * 16 + lax.axis_index("s")

    unroll = 5

    def insert_one(c, t):
        v = row_v[pl.ds(c * 16, 16)]
        t = list(t)
        cur = v
        for i in range(_K):
            hi = jnp.maximum(t[i], cur)
            cur = jnp.minimum(t[i], cur)
            t[i] = hi
        return tuple(t)

    def chunk_body(c, t):
        for u in range(unroll):
            t = insert_one(c * unroll + u, t)
        return t

    def row_body(j, carry):
        row = wid * rows_per_worker + j
        pltpu.sync_copy(align_hbm.at[row], row_v)
        t0 = tuple(jnp.full((16,), -1.0, f32) for _ in range(_K))
        tfin = lax.fori_loop(0, nchunks // unroll, chunk_body, t0)
        for i in range(_K):
            out_v[pl.ds(j * (16 * _K) + i * 16, 16)] = tfin[i]
        return carry

    lax.fori_loop(0, rows_per_worker, row_body, jnp.int32(0))
    pltpu.sync_copy(out_v, out_hbm.at[wid])


def _finish_kernel(align_ref, ciou_ref, cand_ref, gt_labels_ref,
                   gt_bboxes_ref, gt_dist_ref,
                   bbox_out_ref, cls_out_ref, dist_out_ref, fg_out_ref):
    f32 = jnp.float32
    G, A = align_ref.shape[1], align_ref.shape[2]
    C = cls_out_ref.shape[1]

    align = align_ref[0]
    ciou = ciou_ref[0]
    cand = cand_ref[0]
    gtb = gt_bboxes_ref[0]
    labels = gt_labels_ref[0]
    gdist = gt_dist_ref[0]

    work = cand
    thresh = jnp.full((G, 1), -1.0, f32)
    for _ in range(_K):
        thresh = jnp.max(work, axis=1, keepdims=True)
        work = jnp.where(work == thresh, -1.0, work)

    keep10 = (align >= thresh) & (align > 0.0)
    ov_m = jnp.where(keep10, ciou, 0.0)
    al_m = jnp.where(keep10, align, 0.0)

    best_ov = jnp.max(ov_m, axis=0, keepdims=True)
    iota_g = lax.broadcasted_iota(jnp.int32, (G, A), 0)
    best_g = jnp.min(jnp.where(ov_m == best_ov, iota_g, G), axis=0, keepdims=True)
    matched = best_ov > 0.0
    onehot_a = (iota_g == best_g).astype(f32)

    gvals = jnp.concatenate(
        [gtb, gdist, jnp.zeros((G, 3), f32)], axis=1)
    gathered = lax.dot_general(
        gvals, onehot_a, (((0,), (0,)), ((), ())),
        preferred_element_type=f32, precision=lax.Precision.HIGHEST)

    max_al = jnp.max(al_m, axis=1, keepdims=True)
    max_ov = jnp.max(ov_m, axis=1, keepdims=True)
    ratio = max_ov / (max_al + _EPSILON)
    norm = jnp.max(al_m * ratio, axis=0, keepdims=True)

    neg1 = jnp.float32(-1.0)
    bbox_out_ref[0] = jnp.where(matched, gathered[0:4, :], neg1)

    cls = jnp.maximum(labels, 0)
    onehot_cls = (lax.broadcasted_iota(jnp.int32, (G, C), 1) == cls).astype(f32)
    sel_scaled = onehot_a * jnp.where(matched, norm, 0.0)
    cls_out_ref[0] = lax.dot_general(
        onehot_cls, sel_scaled, (((0,), (0,)), ((), ())),
        preferred_element_type=f32, precision=lax.Precision.HIGHEST)

    dist_out_ref[0] = jnp.where(matched, gathered[4:5, :], neg1) * norm
    fg_out_ref[0] = jnp.ones((1, A), f32)


def kernel(scores, decode_bboxes, distances, anchors, gt_labels, gt_bboxes,
           gt_distances, gt_mask):
    del distances
    B, A, C = scores.shape
    G = gt_labels.shape[1]

    scores_t = jnp.transpose(scores, (0, 2, 1))
    decode_t = jnp.transpose(decode_bboxes, (0, 2, 1))
    anchors_t = jnp.transpose(anchors, (1, 0))
    labels3 = gt_labels.reshape(B, G, 1)
    gdist3 = gt_distances.reshape(B, G, 1)
    gmask3 = gt_mask.astype(jnp.float32)

    def row_spec(shape):
        return pl.BlockSpec((1,) + shape, lambda b: (b, 0, 0))

    def sub_pipeline(scores_t, decode_t, labels3, gtb, gmask3, gdist3):
        nb = scores_t.shape[0]

        align, ciou = pl.pallas_call(
            _dense_kernel,
            grid=(nb,),
            in_specs=[
                row_spec((C, A)),
                row_spec((4, A)),
                pl.BlockSpec((2, A), lambda b: (0, 0)),
                row_spec((G, 1)),
                row_spec((G, 4)),
                row_spec((G, 1)),
            ],
            out_specs=(row_spec((G, A)), row_spec((G, A))),
            out_shape=(
                jax.ShapeDtypeStruct((nb, G, A), jnp.float32),
                jax.ShapeDtypeStruct((nb, G, A), jnp.float32),
            ),
        )(scores_t, decode_t, anchors_t, labels3, gtb, gmask3)

        rpw = nb * G // 32
        mesh = plsc.VectorSubcoreMesh(core_axis_name="c",
                                      subcore_axis_name="s")
        sc_call = functools.partial(
            pl.kernel, mesh=mesh,
            out_type=jax.ShapeDtypeStruct((32, rpw * 16 * _K), jnp.float32),
            scratch_types=[
                pltpu.VMEM((A,), jnp.float32),
                pltpu.VMEM((rpw * 16 * _K,), jnp.float32),
            ],
        )(_sc_cand_body)
        cand = sc_call(align.reshape(nb * G, A))
        cand3 = cand.reshape(nb, G, 16 * _K)

        out_shapes = (
            jax.ShapeDtypeStruct((nb, 4, A), jnp.float32),
            jax.ShapeDtypeStruct((nb, C, A), jnp.float32),
            jax.ShapeDtypeStruct((nb, 1, A), jnp.float32),
            jax.ShapeDtypeStruct((nb, 1, A), jnp.float32),
        )
        return pl.pallas_call(
            _finish_kernel,
            grid=(nb,),
            in_specs=[
                row_spec((G, A)),
                row_spec((G, A)),
                row_spec((G, 16 * _K)),
                row_spec((G, 1)),
                row_spec((G, 4)),
                row_spec((G, 1)),
            ],
            out_specs=(row_spec((4, A)), row_spec((C, A)),
                       row_spec((1, A)), row_spec((1, A))),
            out_shape=out_shapes,
        )(align, ciou, cand3, labels3, gtb, gdist3)

    nh = 2 if (B % 2 == 0 and (B // 2) * G % 32 == 0) else 1
    h = B // nh
    parts = [
        sub_pipeline(scores_t[i * h:(i + 1) * h], decode_t[i * h:(i + 1) * h],
                     labels3[i * h:(i + 1) * h], gt_bboxes[i * h:(i + 1) * h],
                     gmask3[i * h:(i + 1) * h], gdist3[i * h:(i + 1) * h])
        for i in range(nh)
    ]
    bbox_l, cls_oh, dist_l, fg = (
        parts[0] if nh == 1 else
        tuple(jnp.concatenate(x, axis=0) for x in zip(*parts)))

    bbox_labels = jnp.transpose(bbox_l, (0, 2, 1))
    class_labels_oh = jnp.transpose(cls_oh, (0, 2, 1))
    dist_labels = dist_l.reshape(B, A)
    fg_mask = fg.reshape(B, A)
    return bbox_labels, class_labels_oh, dist_labels, fg_mask
